# baseline (speedup 1.0000x reference)
import functools

import jax
import jax.numpy as jnp
import numpy as np
from jax import lax
from jax.experimental import pallas as pl
from jax.experimental.pallas import tpu as pltpu
from jax.experimental.pallas import tpu_sc as plsc

_NC_TABLE = 100000
_G = 3
_SP = 0.2
_SN = 0.8

_U32 = np.uint32


def _tf_rounds(x0, x1, ks0, ks1, ks2):
    rot = ((13, 15, 26, 6), (17, 29, 16, 24))
    adds = ((ks1, ks2), (ks2, ks0), (ks0, ks1), (ks1, ks2), (ks2, ks0))
    x0 = (x0 + ks0).astype(_U32)
    x1 = (x1 + ks1).astype(_U32)
    for i in range(5):
        for r in rot[i % 2]:
            x0 = (x0 + x1).astype(_U32)
            x1 = ((x1 << _U32(r)) | (x1 >> _U32(32 - r))).astype(_U32)
            x1 = x0 ^ x1
        a, b = adds[i]
        x0 = (x0 + a).astype(_U32)
        x1 = (x1 + b + _U32(i + 1)).astype(_U32)
    return x0, x1


def _tf_hash(key, x0, x1):
    k0, k1 = _U32(key[0]), _U32(key[1])
    k2 = k0 ^ k1 ^ _U32(0x1BD11BDA)
    return _tf_rounds(x0.astype(_U32), x1.astype(_U32), k0, k1, k2)


def _tf_fold_in(key, data):
    o0, o1 = _tf_hash(key, np.array([data >> 32], np.uint64).astype(_U32),
                      np.array([data & 0xFFFFFFFF], np.uint64).astype(_U32))
    return (o0[0], o1[0])


def _tf_split(key, n):
    o0, o1 = _tf_hash(key, np.zeros(n, _U32), np.arange(n, dtype=_U32))
    return [(o0[i], o1[i]) for i in range(n)]


def _tf_bits32(key, size):
    i64 = np.arange(size, dtype=np.uint64)
    b0, b1 = _tf_hash(key, (i64 >> np.uint64(32)).astype(_U32),
                      (i64 & np.uint64(0xFFFFFFFF)).astype(_U32))
    return b0 ^ b1


def _np_uniform(key, size, lo, hi):
    bits = _tf_bits32(key, size)
    fb = (bits >> _U32(9)) | _U32(0x3F800000)
    f = fb.view(np.float32) - np.float32(1.0)
    lo32, hi32 = np.float32(lo), np.float32(hi)
    return np.maximum(lo32, f * (hi32 - lo32) + lo32)


def _np_normal(key, size):
    from scipy.special import erfinv
    lo = np.nextafter(np.float32(-1.0), np.float32(0.0), dtype=np.float32)
    u = _np_uniform(key, size, lo, np.float32(1.0))
    return (np.float32(np.sqrt(2)) * erfinv(u.astype(np.float64))).astype(np.float32)


_CONST_CACHE = {}


def _denoise_consts(BT, S):
    ck = (BT, S)
    if ck in _CONST_CACHE:
        return _CONST_CACHE[ck]
    N = BT * S
    base = (_U32(0), _U32(42))
    noise6 = np.zeros((2 * _G, 8, N), np.float32)
    fp = np.empty((1 + _G, BT, S), np.int32)
    fp[0] = np.arange(N, dtype=np.int32).reshape(BT, S)
    for g in range(_G):
        kg = _tf_fold_in(base, g)
        k1, k2, k3 = _tf_split(kg, 3)
        noise6[2 * g, :3] = _np_normal(k1, N * 3).reshape(N, 3).T
        noise6[2 * g + 1, :3] = _np_normal(k2, N * 3).reshape(N, 3).T
        u = _np_uniform(k3, N, 0.0, 1.0).reshape(BT, S)
        perm = np.argsort(u, axis=-1, kind="stable").astype(np.int32)
        fp[1 + g] = fp[0, :, 0:1] + perm
    _CONST_CACHE[ck] = (noise6, fp.reshape(-1))
    return _CONST_CACHE[ck]


_CHUNK = 128


def _sc_gather(table, cls_flat, fp_flat, n_rows, d):
    info = plsc.get_sparse_core_info()
    nw = info.num_cores * info.num_subcores
    rows_w = n_rows // nw
    chunks_w = rows_w // _CHUNK
    assert n_rows % (nw * _CHUNK) == 0
    n_src = cls_flat.shape[0]

    nbuf = 4

    @functools.partial(
        pl.kernel,
        mesh=plsc.VectorSubcoreMesh(core_axis_name="c", subcore_axis_name="s"),
        compiler_params=pltpu.CompilerParams(needs_layout_passes=False),
        out_type=(jax.ShapeDtypeStruct((n_rows, d), jnp.float32),
                  jax.ShapeDtypeStruct((n_rows,), jnp.int32)),
        scratch_types=(
            [pltpu.VMEM((rows_w,), jnp.int32)]
            + [pltpu.VMEM((_CHUNK,), jnp.int32) for _ in range(nbuf)]
            + [pltpu.VMEM((_CHUNK, d), jnp.float32) for _ in range(nbuf)]
            + [pltpu.SemaphoreType.DMA for _ in range(4 * nbuf)]
        ),
    )
    def gather_kernel(table_hbm, cls_hbm, fp_hbm, out_hbm, idxout_hbm,
                      fp_v, *bufs_sems):
        ibufs = bufs_sems[:nbuf]
        rbufs = bufs_sems[nbuf:2 * nbuf]
        isems = bufs_sems[2 * nbuf:3 * nbuf]
        gsems = bufs_sems[3 * nbuf:4 * nbuf]
        osems = bufs_sems[4 * nbuf:5 * nbuf]
        xsems = bufs_sems[5 * nbuf:6 * nbuf]
        wid = lax.axis_index("s") * info.num_cores + lax.axis_index("c")
        rbase = wid * rows_w
        pltpu.sync_copy(fp_hbm.at[pl.ds(rbase, rows_w)], fp_v)

        def start_i(j):
            return pltpu.async_copy(
                cls_hbm.at[fp_v.at[pl.ds(j * _CHUNK, _CHUNK)]],
                ibufs[j % nbuf], isems[j % nbuf])

        hi = [None] * chunks_w
        hg = [None] * chunks_w
        ho = [None] * chunks_w
        hx = [None] * chunks_w
        for j in range(min(nbuf, chunks_w)):
            hi[j] = start_i(j)
        for j in range(chunks_w):
            s = j % nbuf
            hi[j].wait()
            hx[j] = pltpu.async_copy(
                ibufs[s], idxout_hbm.at[pl.ds(rbase + j * _CHUNK, _CHUNK)],
                xsems[s])
            if j >= nbuf:
                ho[j - nbuf].wait()
            hg[j] = pltpu.async_copy(
                table_hbm.at[ibufs[s]], rbufs[s], gsems[s])
            hg[j].wait()
            if j + nbuf < chunks_w:
                hx[j].wait()
                hi[j + nbuf] = start_i(j + nbuf)
            ho[j] = pltpu.async_copy(
                rbufs[s],
                out_hbm.at[pl.ds(rbase + j * _CHUNK, _CHUNK)],
                osems[s])
        for j in range(max(0, chunks_w - nbuf), chunks_w):
            ho[j].wait()
            if hx[j] is not None and j + nbuf >= chunks_w:
                hx[j].wait()

    return gather_kernel(table, cls_flat, fp_flat)


def _tc_units(doa_t, noise6, unit_ref):
    d_ = doa_t[...]
    for j in range(2 * _G):
        sig = _SP if j % 2 == 0 else _SN
        x = d_ + sig * noise6[j]
        n2 = jnp.sum(x * x, axis=0, keepdims=True)
        inv = 1.0 / jnp.maximum(jnp.sqrt(n2), 1e-12)
        unit_ref[j] = x * inv


def _tc_main(e4, unit6, mask_t, pw, dwpad, bias, dn_ref):
    f32 = jnp.float32
    pw_ = pw[...]
    w2 = jnp.dot(pw_, dwpad[...], preferred_element_type=f32)
    bb = bias[...]
    m_col = jnp.transpose(mask_t[0:1, :])
    projs = []
    for k in range(4):
        projs.append(lax.dot_general(
            e4[k], pw_, (((1,), (1,)), ((), ())),
            preferred_element_type=f32))
    rb = dn_ref.shape[0]
    s = dn_ref.shape[1] // (2 * _G)
    dcap = dn_ref.shape[2]
    for g in range(_G):
        for sgn in range(2):
            j = 2 * g + sgn
            contrib = lax.dot_general(
                unit6[j], w2, (((0,), (1,)), ((), ())),
                preferred_element_type=f32)
            src = 0 if sgn == 0 else g + 1
            q = (projs[src] + contrib + bb) * m_col
            dn_ref[:, j * s:(j + 1) * s, :] = q.reshape(rb, s, dcap)


def kernel(gt_cls, gt_doa, gt_loud, gt_mask, class_embed, doa_w, proj_w, proj_b):
    B, T, S = gt_cls.shape
    BT = B * T
    N = BT * S
    D = class_embed.shape[1]

    noise6_np, fp_np = _denoise_consts(BT, S)
    noise6 = jnp.asarray(noise6_np)
    fp_flat = jnp.asarray(fp_np)

    cls_flat = gt_cls.reshape(N).astype(jnp.int32)
    cls_safe = jnp.where(cls_flat < 0, _NC_TABLE, cls_flat)

    gathered, idx_out = _sc_gather(class_embed, cls_safe, fp_flat, 4 * N, D)
    g4 = gathered.reshape(4, N, D)

    doa_t = jnp.pad(jnp.transpose(gt_doa.reshape(N, 3)), ((0, 5), (0, 0)))
    mask_t = jnp.broadcast_to(
        gt_mask.reshape(N).astype(jnp.float32)[None, :], (8, N))
    dw_pad = jnp.pad(doa_w, ((0, 0), (0, 5)))
    bias2 = proj_b.reshape(1, D)

    RU = 4096
    unit6 = pl.pallas_call(
        _tc_units,
        grid=(N // RU,),
        in_specs=[
            pl.BlockSpec((8, RU), lambda i: (0, i)),
            pl.BlockSpec((2 * _G, 8, RU), lambda i: (0, 0, i)),
        ],
        out_specs=pl.BlockSpec((2 * _G, 8, RU), lambda i: (0, 0, i)),
        out_shape=jax.ShapeDtypeStruct((2 * _G, 8, N), jnp.float32),
    )(doa_t, noise6)

    RB = 64
    nblk = BT // RB
    R = RB * S

    dn = pl.pallas_call(
        _tc_main,
        grid=(nblk,),
        in_specs=[
            pl.BlockSpec((4, R, D), lambda i: (0, i, 0)),
            pl.BlockSpec((2 * _G, 8, R), lambda i: (0, 0, i)),
            pl.BlockSpec((8, R), lambda i: (0, i)),
            pl.BlockSpec((D, D), lambda i: (0, 0)),
            pl.BlockSpec((D, 8), lambda i: (0, 0)),
            pl.BlockSpec((1, D), lambda i: (0, 0)),
        ],
        out_specs=pl.BlockSpec((RB, 2 * _G * S, D), lambda i: (i, 0, 0)),
        out_shape=jax.ShapeDtypeStruct((BT, 2 * _G * S, D), jnp.float32),
    )(g4, unit6, mask_t, proj_w, dw_pad, bias2)

    u3 = unit6[:, :3, :].reshape(2 * _G, 3, BT, S)
    pos_doa = u3[0::2].transpose(2, 0, 3, 1)
    neg_doa = u3[1::2].transpose(2, 0, 3, 1)
    pos_cls = jnp.broadcast_to(
        cls_safe.reshape(BT, S)[:, None, :], (BT, _G, S))
    neg_cls = idx_out[N:].reshape(_G, BT, S).transpose(1, 0, 2)
    return (dn, pos_cls, pos_doa, neg_cls, neg_doa)


# speedup vs baseline: 1.0739x; 1.0739x over previous
import functools

import jax
import jax.numpy as jnp
import numpy as np
from jax import lax
from jax.experimental import pallas as pl
from jax.experimental.pallas import tpu as pltpu
from jax.experimental.pallas import tpu_sc as plsc

_NC_TABLE = 100000
_G = 3
_SP = 0.2
_SN = 0.8

_U32 = np.uint32


def _tf_rounds(x0, x1, ks0, ks1, ks2):
    rot = ((13, 15, 26, 6), (17, 29, 16, 24))
    adds = ((ks1, ks2), (ks2, ks0), (ks0, ks1), (ks1, ks2), (ks2, ks0))
    x0 = (x0 + ks0).astype(_U32)
    x1 = (x1 + ks1).astype(_U32)
    for i in range(5):
        for r in rot[i % 2]:
            x0 = (x0 + x1).astype(_U32)
            x1 = ((x1 << _U32(r)) | (x1 >> _U32(32 - r))).astype(_U32)
            x1 = x0 ^ x1
        a, b = adds[i]
        x0 = (x0 + a).astype(_U32)
        x1 = (x1 + b + _U32(i + 1)).astype(_U32)
    return x0, x1


def _tf_hash(key, x0, x1):
    k0, k1 = _U32(key[0]), _U32(key[1])
    k2 = k0 ^ k1 ^ _U32(0x1BD11BDA)
    return _tf_rounds(x0.astype(_U32), x1.astype(_U32), k0, k1, k2)


def _tf_fold_in(key, data):
    o0, o1 = _tf_hash(key, np.array([data >> 32], np.uint64).astype(_U32),
                      np.array([data & 0xFFFFFFFF], np.uint64).astype(_U32))
    return (o0[0], o1[0])


def _tf_split(key, n):
    o0, o1 = _tf_hash(key, np.zeros(n, _U32), np.arange(n, dtype=_U32))
    return [(o0[i], o1[i]) for i in range(n)]


def _tf_bits32(key, size):
    i64 = np.arange(size, dtype=np.uint64)
    b0, b1 = _tf_hash(key, (i64 >> np.uint64(32)).astype(_U32),
                      (i64 & np.uint64(0xFFFFFFFF)).astype(_U32))
    return b0 ^ b1


def _np_uniform(key, size, lo, hi):
    bits = _tf_bits32(key, size)
    fb = (bits >> _U32(9)) | _U32(0x3F800000)
    f = fb.view(np.float32) - np.float32(1.0)
    lo32, hi32 = np.float32(lo), np.float32(hi)
    return np.maximum(lo32, f * (hi32 - lo32) + lo32)


def _np_normal(key, size):
    from scipy.special import erfinv
    lo = np.nextafter(np.float32(-1.0), np.float32(0.0), dtype=np.float32)
    u = _np_uniform(key, size, lo, np.float32(1.0))
    return (np.float32(np.sqrt(2)) * erfinv(u.astype(np.float64))).astype(np.float32)


_CONST_CACHE = {}


def _denoise_consts(BT, S):
    ck = (BT, S)
    if ck in _CONST_CACHE:
        return _CONST_CACHE[ck]
    N = BT * S
    base = (_U32(0), _U32(42))
    noise6 = np.zeros((2 * _G, 8, N), np.float32)
    fp = np.empty((1 + _G, BT, S), np.int32)
    fp[0] = np.arange(N, dtype=np.int32).reshape(BT, S)
    for g in range(_G):
        kg = _tf_fold_in(base, g)
        k1, k2, k3 = _tf_split(kg, 3)
        noise6[2 * g, :3] = _np_normal(k1, N * 3).reshape(N, 3).T
        noise6[2 * g + 1, :3] = _np_normal(k2, N * 3).reshape(N, 3).T
        u = _np_uniform(k3, N, 0.0, 1.0).reshape(BT, S)
        perm = np.argsort(u, axis=-1, kind="stable").astype(np.int32)
        fp[1 + g] = fp[0, :, 0:1] + perm
    _CONST_CACHE[ck] = (noise6, fp.reshape(-1))
    return _CONST_CACHE[ck]


_CHUNK = 128


def _sc_gather(table, cls_flat, fp_flat, n_rows, d):
    info = plsc.get_sparse_core_info()
    nw = info.num_cores * info.num_subcores
    rows_w = n_rows // nw
    chunks_w = rows_w // _CHUNK
    assert n_rows % (nw * _CHUNK) == 0
    n_src = cls_flat.shape[0]

    nbuf = 4

    @functools.partial(
        pl.kernel,
        mesh=plsc.VectorSubcoreMesh(core_axis_name="c", subcore_axis_name="s"),
        compiler_params=pltpu.CompilerParams(needs_layout_passes=False),
        out_type=(jax.ShapeDtypeStruct((n_rows, d), jnp.float32),
                  jax.ShapeDtypeStruct((n_rows,), jnp.int32)),
        scratch_types=(
            [pltpu.VMEM((rows_w,), jnp.int32),
             pltpu.VMEM((rows_w,), jnp.int32)]
            + [pltpu.VMEM((_CHUNK, d), jnp.float32) for _ in range(nbuf)]
            + [pltpu.SemaphoreType.DMA,
               pltpu.SemaphoreType.DMA]
            + [pltpu.SemaphoreType.DMA for _ in range(2 * nbuf)]
        ),
    )
    def gather_kernel(table_hbm, cls_hbm, fp_hbm, out_hbm, idxout_hbm,
                      fp_v, idx_v, *bufs_sems):
        rbufs = bufs_sems[:nbuf]
        isem = bufs_sems[nbuf]
        xsem = bufs_sems[nbuf + 1]
        gsems = bufs_sems[nbuf + 2:2 * nbuf + 2]
        osems = bufs_sems[2 * nbuf + 2:3 * nbuf + 2]
        wid = lax.axis_index("s") * info.num_cores + lax.axis_index("c")
        rbase = wid * rows_w
        pltpu.sync_copy(fp_hbm.at[pl.ds(rbase, rows_w)], fp_v)

        hi = [pltpu.async_copy(
            cls_hbm.at[fp_v.at[pl.ds(j * _CHUNK, _CHUNK)]],
            idx_v.at[pl.ds(j * _CHUNK, _CHUNK)], isem)
            for j in range(chunks_w)]
        for h in hi:
            h.wait()
        hx = pltpu.async_copy(
            idx_v, idxout_hbm.at[pl.ds(rbase, rows_w)], xsem)

        def start_g(j):
            return pltpu.async_copy(
                table_hbm.at[idx_v.at[pl.ds(j * _CHUNK, _CHUNK)]],
                rbufs[j % nbuf], gsems[j % nbuf])

        hg = [None] * chunks_w
        ho = [None] * chunks_w
        for j in range(min(nbuf - 1, chunks_w)):
            hg[j] = start_g(j)
        for j in range(chunks_w):
            nxt = j + nbuf - 1
            if nxt < chunks_w:
                if nxt >= nbuf:
                    ho[nxt - nbuf].wait()
                hg[nxt] = start_g(nxt)
            hg[j].wait()
            ho[j] = pltpu.async_copy(
                rbufs[j % nbuf],
                out_hbm.at[pl.ds(rbase + j * _CHUNK, _CHUNK)],
                osems[j % nbuf])
        for j in range(max(0, chunks_w - nbuf), chunks_w):
            ho[j].wait()
        hx.wait()

    return gather_kernel(table, cls_flat, fp_flat)


def _tc_units(doa_t, noise6, unit_ref):
    d_ = doa_t[...]
    for j in range(2 * _G):
        sig = _SP if j % 2 == 0 else _SN
        x = d_ + sig * noise6[j]
        n2 = jnp.sum(x * x, axis=0, keepdims=True)
        inv = 1.0 / jnp.maximum(jnp.sqrt(n2), 1e-12)
        unit_ref[j] = x * inv


def _tc_main(e4, unit6, mask_t, pw, dwpad, bias, dn_ref):
    f32 = jnp.float32
    pw_ = pw[...]
    w2 = jnp.dot(pw_, dwpad[...], preferred_element_type=f32)
    bb = bias[...]
    m_col = jnp.transpose(mask_t[0:1, :])
    projs = []
    for k in range(4):
        projs.append(lax.dot_general(
            e4[k], pw_, (((1,), (1,)), ((), ())),
            preferred_element_type=f32))
    rb = dn_ref.shape[0]
    s = dn_ref.shape[1] // (2 * _G)
    dcap = dn_ref.shape[2]
    for g in range(_G):
        for sgn in range(2):
            j = 2 * g + sgn
            contrib = lax.dot_general(
                unit6[j], w2, (((0,), (1,)), ((), ())),
                preferred_element_type=f32)
            src = 0 if sgn == 0 else g + 1
            q = (projs[src] + contrib + bb) * m_col
            dn_ref[:, j * s:(j + 1) * s, :] = q.reshape(rb, s, dcap)


def kernel(gt_cls, gt_doa, gt_loud, gt_mask, class_embed, doa_w, proj_w, proj_b):
    B, T, S = gt_cls.shape
    BT = B * T
    N = BT * S
    D = class_embed.shape[1]

    noise6_np, fp_np = _denoise_consts(BT, S)
    noise6 = jnp.asarray(noise6_np)
    fp_flat = jnp.asarray(fp_np)

    cls_flat = gt_cls.reshape(N).astype(jnp.int32)
    cls_safe = jnp.where(cls_flat < 0, _NC_TABLE, cls_flat)

    gathered, idx_out = _sc_gather(class_embed, cls_safe, fp_flat, 4 * N, D)
    g4 = gathered.reshape(4, N, D)

    doa_t = jnp.pad(jnp.transpose(gt_doa.reshape(N, 3)), ((0, 5), (0, 0)))
    mask_t = jnp.broadcast_to(
        gt_mask.reshape(N).astype(jnp.float32)[None, :], (8, N))
    dw_pad = jnp.pad(doa_w, ((0, 0), (0, 5)))
    bias2 = proj_b.reshape(1, D)

    RU = 4096
    unit6 = pl.pallas_call(
        _tc_units,
        grid=(N // RU,),
        in_specs=[
            pl.BlockSpec((8, RU), lambda i: (0, i)),
            pl.BlockSpec((2 * _G, 8, RU), lambda i: (0, 0, i)),
        ],
        out_specs=pl.BlockSpec((2 * _G, 8, RU), lambda i: (0, 0, i)),
        out_shape=jax.ShapeDtypeStruct((2 * _G, 8, N), jnp.float32),
    )(doa_t, noise6)

    RB = 64
    nblk = BT // RB
    R = RB * S

    dn = pl.pallas_call(
        _tc_main,
        grid=(nblk,),
        in_specs=[
            pl.BlockSpec((4, R, D), lambda i: (0, i, 0)),
            pl.BlockSpec((2 * _G, 8, R), lambda i: (0, 0, i)),
            pl.BlockSpec((8, R), lambda i: (0, i)),
            pl.BlockSpec((D, D), lambda i: (0, 0)),
            pl.BlockSpec((D, 8), lambda i: (0, 0)),
            pl.BlockSpec((1, D), lambda i: (0, 0)),
        ],
        out_specs=pl.BlockSpec((RB, 2 * _G * S, D), lambda i: (i, 0, 0)),
        out_shape=jax.ShapeDtypeStruct((BT, 2 * _G * S, D), jnp.float32),
    )(g4, unit6, mask_t, proj_w, dw_pad, bias2)

    u3 = unit6[:, :3, :].reshape(2 * _G, 3, BT, S)
    pos_doa = u3[0::2].transpose(2, 0, 3, 1)
    neg_doa = u3[1::2].transpose(2, 0, 3, 1)
    pos_cls = jnp.broadcast_to(
        cls_safe.reshape(BT, S)[:, None, :], (BT, _G, S))
    neg_cls = idx_out[N:].reshape(_G, BT, S).transpose(1, 0, 2)
    return (dn, pos_cls, pos_doa, neg_cls, neg_doa)


# speedup vs baseline: 1.1165x; 1.0397x over previous
import functools

import jax
import jax.numpy as jnp
import numpy as np
from jax import lax
from jax.experimental import pallas as pl
from jax.experimental.pallas import tpu as pltpu
from jax.experimental.pallas import tpu_sc as plsc

_NC_TABLE = 100000
_G = 3
_SP = 0.2
_SN = 0.8

_U32 = np.uint32


def _tf_rounds(x0, x1, ks0, ks1, ks2):
    rot = ((13, 15, 26, 6), (17, 29, 16, 24))
    adds = ((ks1, ks2), (ks2, ks0), (ks0, ks1), (ks1, ks2), (ks2, ks0))
    x0 = (x0 + ks0).astype(_U32)
    x1 = (x1 + ks1).astype(_U32)
    for i in range(5):
        for r in rot[i % 2]:
            x0 = (x0 + x1).astype(_U32)
            x1 = ((x1 << _U32(r)) | (x1 >> _U32(32 - r))).astype(_U32)
            x1 = x0 ^ x1
        a, b = adds[i]
        x0 = (x0 + a).astype(_U32)
        x1 = (x1 + b + _U32(i + 1)).astype(_U32)
    return x0, x1


def _tf_hash(key, x0, x1):
    k0, k1 = _U32(key[0]), _U32(key[1])
    k2 = k0 ^ k1 ^ _U32(0x1BD11BDA)
    return _tf_rounds(x0.astype(_U32), x1.astype(_U32), k0, k1, k2)


def _tf_fold_in(key, data):
    o0, o1 = _tf_hash(key, np.array([data >> 32], np.uint64).astype(_U32),
                      np.array([data & 0xFFFFFFFF], np.uint64).astype(_U32))
    return (o0[0], o1[0])


def _tf_split(key, n):
    o0, o1 = _tf_hash(key, np.zeros(n, _U32), np.arange(n, dtype=_U32))
    return [(o0[i], o1[i]) for i in range(n)]


def _tf_bits32(key, size):
    i64 = np.arange(size, dtype=np.uint64)
    b0, b1 = _tf_hash(key, (i64 >> np.uint64(32)).astype(_U32),
                      (i64 & np.uint64(0xFFFFFFFF)).astype(_U32))
    return b0 ^ b1


def _np_uniform(key, size, lo, hi):
    bits = _tf_bits32(key, size)
    fb = (bits >> _U32(9)) | _U32(0x3F800000)
    f = fb.view(np.float32) - np.float32(1.0)
    lo32, hi32 = np.float32(lo), np.float32(hi)
    return np.maximum(lo32, f * (hi32 - lo32) + lo32)


def _np_normal(key, size):
    from scipy.special import erfinv
    lo = np.nextafter(np.float32(-1.0), np.float32(0.0), dtype=np.float32)
    u = _np_uniform(key, size, lo, np.float32(1.0))
    return (np.float32(np.sqrt(2)) * erfinv(u.astype(np.float64))).astype(np.float32)


_CONST_CACHE = {}


def _denoise_consts(BT, S):
    ck = (BT, S)
    if ck in _CONST_CACHE:
        return _CONST_CACHE[ck]
    N = BT * S
    base = (_U32(0), _U32(42))
    noise6 = np.zeros((2 * _G, 8, N), np.float32)
    fp = np.empty((1 + _G, BT, S), np.int32)
    fp[0] = np.arange(N, dtype=np.int32).reshape(BT, S)
    for g in range(_G):
        kg = _tf_fold_in(base, g)
        k1, k2, k3 = _tf_split(kg, 3)
        noise6[2 * g, :3] = _np_normal(k1, N * 3).reshape(N, 3).T
        noise6[2 * g + 1, :3] = _np_normal(k2, N * 3).reshape(N, 3).T
        u = _np_uniform(k3, N, 0.0, 1.0).reshape(BT, S)
        perm = np.argsort(u, axis=-1, kind="stable").astype(np.int32)
        fp[1 + g] = fp[0, :, 0:1] + perm
    _CONST_CACHE[ck] = (noise6, fp.reshape(-1))
    return _CONST_CACHE[ck]


_CHUNK = 128


def _sc_gather(table, cls_flat, fp_flat, n_rows, d):
    info = plsc.get_sparse_core_info()
    nw = info.num_cores * info.num_subcores
    rows_w = n_rows // nw
    chunks_w = rows_w // _CHUNK
    assert n_rows % (nw * _CHUNK) == 0
    n_src = cls_flat.shape[0]

    nbuf = 4

    @functools.partial(
        pl.kernel,
        mesh=plsc.VectorSubcoreMesh(core_axis_name="c", subcore_axis_name="s"),
        compiler_params=pltpu.CompilerParams(needs_layout_passes=False),
        out_type=(jax.ShapeDtypeStruct((n_rows, d), jnp.float32),
                  jax.ShapeDtypeStruct((n_rows,), jnp.int32)),
        scratch_types=(
            [pltpu.VMEM((rows_w,), jnp.int32),
             pltpu.VMEM((rows_w,), jnp.int32)]
            + [pltpu.VMEM((_CHUNK, d), jnp.float32) for _ in range(nbuf)]
            + [pltpu.SemaphoreType.DMA,
               pltpu.SemaphoreType.DMA]
            + [pltpu.SemaphoreType.DMA for _ in range(2 * nbuf)]
        ),
    )
    def gather_kernel(table_hbm, cls_hbm, fp_hbm, out_hbm, idxout_hbm,
                      fp_v, idx_v, *bufs_sems):
        rbufs = bufs_sems[:nbuf]
        isem = bufs_sems[nbuf]
        xsem = bufs_sems[nbuf + 1]
        gsems = bufs_sems[nbuf + 2:2 * nbuf + 2]
        osems = bufs_sems[2 * nbuf + 2:3 * nbuf + 2]
        wid = lax.axis_index("s") * info.num_cores + lax.axis_index("c")
        rbase = wid * rows_w
        pltpu.sync_copy(fp_hbm.at[pl.ds(rbase, rows_w)], fp_v)

        hi = [pltpu.async_copy(
            cls_hbm.at[fp_v.at[pl.ds(j * _CHUNK, _CHUNK)]],
            idx_v.at[pl.ds(j * _CHUNK, _CHUNK)], isem)
            for j in range(chunks_w)]
        for h in hi:
            h.wait()
        hx = pltpu.async_copy(
            idx_v, idxout_hbm.at[pl.ds(rbase, rows_w)], xsem)

        def start_g(j):
            return pltpu.async_copy(
                table_hbm.at[idx_v.at[pl.ds(j * _CHUNK, _CHUNK)]],
                rbufs[j % nbuf], gsems[j % nbuf])

        hg = [None] * chunks_w
        ho = [None] * chunks_w
        for j in range(min(nbuf - 1, chunks_w)):
            hg[j] = start_g(j)
        for j in range(chunks_w):
            nxt = j + nbuf - 1
            if nxt < chunks_w:
                if nxt >= nbuf:
                    ho[nxt - nbuf].wait()
                hg[nxt] = start_g(nxt)
            hg[j].wait()
            ho[j] = pltpu.async_copy(
                rbufs[j % nbuf],
                out_hbm.at[pl.ds(rbase + j * _CHUNK, _CHUNK)],
                osems[j % nbuf])
        for j in range(max(0, chunks_w - nbuf), chunks_w):
            ho[j].wait()
        hx.wait()

    return gather_kernel(table, cls_flat, fp_flat)


def _tc_units(doa_t, noise6, unit_ref):
    d_ = doa_t[...]
    for j in range(2 * _G):
        sig = _SP if j % 2 == 0 else _SN
        x = d_ + sig * noise6[j]
        n2 = jnp.sum(x * x, axis=0, keepdims=True)
        inv = 1.0 / jnp.maximum(jnp.sqrt(n2), 1e-12)
        unit_ref[j] = x * inv


def _tc_main(e4, unit6, mask_t, pw, dwpad, bias, dn_ref):
    f32 = jnp.float32
    pw_ = pw[...]
    w2 = jnp.dot(pw_, dwpad[...], preferred_element_type=f32)
    bb = bias[...]
    m_col = jnp.transpose(mask_t[0:1, :])
    projs = []
    for k in range(4):
        projs.append(lax.dot_general(
            e4[k], pw_, (((1,), (1,)), ((), ())),
            preferred_element_type=f32))
    rb = dn_ref.shape[0]
    s = dn_ref.shape[1] // (2 * _G)
    dcap = dn_ref.shape[2]
    for g in range(_G):
        for sgn in range(2):
            j = 2 * g + sgn
            contrib = lax.dot_general(
                unit6[j], w2, (((0,), (1,)), ((), ())),
                preferred_element_type=f32)
            src = 0 if sgn == 0 else g + 1
            q = (projs[src] + contrib + bb) * m_col
            dn_ref[:, j * s:(j + 1) * s, :] = q.reshape(rb, s, dcap)


def kernel(gt_cls, gt_doa, gt_loud, gt_mask, class_embed, doa_w, proj_w, proj_b):
    B, T, S = gt_cls.shape
    BT = B * T
    N = BT * S
    D = class_embed.shape[1]

    noise6_np, fp_np = _denoise_consts(BT, S)
    noise6 = jnp.asarray(noise6_np)
    fp_flat = jnp.asarray(fp_np)

    cls_flat = gt_cls.reshape(N).astype(jnp.int32)
    cls_safe = jnp.where(cls_flat < 0, _NC_TABLE, cls_flat)

    gathered, idx_out = _sc_gather(class_embed, cls_safe, fp_flat, 4 * N, D)
    g4 = gathered.reshape(4, N, D)

    doa_t = jnp.pad(jnp.transpose(gt_doa.reshape(N, 3)), ((0, 5), (0, 0)))
    mask_t = jnp.broadcast_to(
        gt_mask.reshape(N).astype(jnp.float32)[None, :], (8, N))
    dw_pad = jnp.pad(doa_w, ((0, 0), (0, 5)))
    bias2 = proj_b.reshape(1, D)

    RU = 4096
    unit6 = pl.pallas_call(
        _tc_units,
        grid=(N // RU,),
        in_specs=[
            pl.BlockSpec((8, RU), lambda i: (0, i)),
            pl.BlockSpec((2 * _G, 8, RU), lambda i: (0, 0, i)),
        ],
        out_specs=pl.BlockSpec((2 * _G, 8, RU), lambda i: (0, 0, i)),
        out_shape=jax.ShapeDtypeStruct((2 * _G, 8, N), jnp.float32),
    )(doa_t, noise6)

    RB = 128
    nblk = BT // RB
    R = RB * S

    dn = pl.pallas_call(
        _tc_main,
        grid=(nblk,),
        in_specs=[
            pl.BlockSpec((4, R, D), lambda i: (0, i, 0)),
            pl.BlockSpec((2 * _G, 8, R), lambda i: (0, 0, i)),
            pl.BlockSpec((8, R), lambda i: (0, i)),
            pl.BlockSpec((D, D), lambda i: (0, 0)),
            pl.BlockSpec((D, 8), lambda i: (0, 0)),
            pl.BlockSpec((1, D), lambda i: (0, 0)),
        ],
        out_specs=pl.BlockSpec((RB, 2 * _G * S, D), lambda i: (i, 0, 0)),
        out_shape=jax.ShapeDtypeStruct((BT, 2 * _G * S, D), jnp.float32),
    )(g4, unit6, mask_t, proj_w, dw_pad, bias2)

    u3 = unit6[:, :3, :].reshape(2 * _G, 3, BT, S)
    pos_doa = u3[0::2].transpose(2, 0, 3, 1)
    neg_doa = u3[1::2].transpose(2, 0, 3, 1)
    pos_cls = jnp.broadcast_to(
        cls_safe.reshape(BT, S)[:, None, :], (BT, _G, S))
    neg_cls = idx_out[N:].reshape(_G, BT, S).transpose(1, 0, 2)
    return (dn, pos_cls, pos_doa, neg_cls, neg_doa)
